# Initial kernel scaffold; baseline (speedup 1.0000x reference)
import jax
import jax.numpy as jnp
from jax.experimental import pallas as pl


def kernel(inputs, embedding_weight):
    raise NotImplementedError("write your pallas kernel here")


# speedup vs baseline: 3.2733x; 3.2733x over previous
import jax
import jax.numpy as jnp
from jax.experimental import pallas as pl


def _add_block(x_ref, w_ref, o_ref):
    o_ref[...] = x_ref[...] + w_ref[...][None, ...]


def kernel(inputs, embedding_weight):
    batch, seq_len, model_dim = inputs.shape
    block_s = 256
    grid = (seq_len // block_s,)
    return pl.pallas_call(
        _add_block,
        grid=grid,
        in_specs=[
            pl.BlockSpec((batch, block_s, model_dim), lambda i: (0, i, 0)),
            pl.BlockSpec((block_s, model_dim), lambda i: (i, 0)),
        ],
        out_specs=pl.BlockSpec((batch, block_s, model_dim), lambda i: (0, i, 0)),
        out_shape=jax.ShapeDtypeStruct((batch, seq_len, model_dim), inputs.dtype),
    )(inputs, embedding_weight)


# speedup vs baseline: 3.2826x; 1.0028x over previous
import jax
import jax.numpy as jnp
from jax.experimental import pallas as pl


def _add_block(x_ref, w_ref, o_ref):
    o_ref[...] = x_ref[...] + w_ref[...][None, ...]


def kernel(inputs, embedding_weight):
    batch, seq_len, model_dim = inputs.shape
    block_s = 512
    grid = (seq_len // block_s,)
    return pl.pallas_call(
        _add_block,
        grid=grid,
        in_specs=[
            pl.BlockSpec((batch, block_s, model_dim), lambda i: (0, i, 0)),
            pl.BlockSpec((block_s, model_dim), lambda i: (i, 0)),
        ],
        out_specs=pl.BlockSpec((batch, block_s, model_dim), lambda i: (0, i, 0)),
        out_shape=jax.ShapeDtypeStruct((batch, seq_len, model_dim), inputs.dtype),
    )(inputs, embedding_weight)
